# Initial kernel scaffold; baseline (speedup 1.0000x reference)
import jax
import jax.numpy as jnp
from jax.experimental import pallas as pl


def kernel(x, edge_index, W1, b1, W2, b2):
    raise NotImplementedError("write your pallas kernel here")


# speedup vs baseline: 10.3888x; 10.3888x over previous
import functools

import jax
import jax.numpy as jnp
from jax import lax
from jax.experimental import pallas as pl
from jax.experimental.pallas import tpu as pltpu
from jax.experimental.pallas import tpu_sc as plsc

N = 10000
E = 320000
D = 128

NC = 2
NS = 16
NW = NC * NS
C = 128
K = -(-E // (NW * C))
E_PAD = NW * K * C
NPAD = 10240
ZPT = NPAD // NS


@functools.cache
def _sc_kernels():
    mesh = plsc.VectorSubcoreMesh(
        core_axis_name="c", subcore_axis_name="s",
        num_cores=NC, num_subcores=NS)

    @functools.partial(
        pl.kernel,
        out_type=jax.ShapeDtypeStruct((NC, NPAD), jnp.float32),
        mesh=mesh,
        scratch_types=[
            pltpu.VMEM_SHARED((NPAD,), jnp.float32),
            pltpu.VMEM((K, C), jnp.int32),
            pltpu.VMEM((C,), jnp.float32),
            pltpu.VMEM((ZPT,), jnp.float32),
        ],
    )
    def hist(dstr_hbm, out_hbm, acc, idx_t, ones_t, zb):
        c = lax.axis_index("c")
        s = lax.axis_index("s")
        wid = c * NS + s

        def z16(i, _):
            zb[pl.ds(i * 16, 16)] = jnp.zeros((16,), jnp.float32)
            return 0
        lax.fori_loop(0, ZPT // 16, z16, 0)

        def o16(i, _):
            ones_t[pl.ds(i * 16, 16)] = jnp.ones((16,), jnp.float32)
            return 0
        lax.fori_loop(0, C // 16, o16, 0)

        pltpu.sync_copy(zb, acc.at[pl.ds(s * ZPT, ZPT)])
        pltpu.sync_copy(dstr_hbm.at[wid], idx_t)
        plsc.subcore_barrier()

        def chunk(j, _):
            pltpu.sync_copy(ones_t, acc.at[idx_t.at[j]], add=True)
            return 0
        lax.fori_loop(0, K, chunk, 0)

        plsc.subcore_barrier()
        pltpu.sync_copy(acc.at[pl.ds(s * ZPT, ZPT)],
                        out_hbm.at[c, pl.ds(s * ZPT, ZPT)])

    @functools.partial(
        pl.kernel,
        out_type=jax.ShapeDtypeStruct((NC, NPAD, D), jnp.float32),
        mesh=mesh,
        scratch_types=[
            pltpu.VMEM_SHARED((NPAD, D), jnp.float32),
            pltpu.VMEM((K, C), jnp.int32),
            pltpu.VMEM((K, C), jnp.int32),
            pltpu.VMEM((C, D), jnp.float32),
            pltpu.SemaphoreType.DMA,
        ],
    )
    def agg(p_hbm, srcr_hbm, dstr_hbm, out_hbm, acc, src_t, dst_t, rows, sem):
        c = lax.axis_index("c")
        s = lax.axis_index("s")
        wid = c * NS + s

        def z16(t, _):
            rows[t // (D // 16), pl.ds((t % (D // 16)) * 16, 16)] = (
                jnp.zeros((16,), jnp.float32))
            return 0
        lax.fori_loop(0, C * (D // 16), z16, 0)

        for r in range(ZPT // C):
            pltpu.sync_copy(rows, acc.at[pl.ds(s * ZPT + r * C, C)])
        pltpu.sync_copy(srcr_hbm.at[wid], src_t)
        pltpu.sync_copy(dstr_hbm.at[wid], dst_t)
        plsc.subcore_barrier()

        def chunk(j, _):
            pltpu.async_copy(p_hbm.at[src_t.at[j]], rows, sem).wait()
            pltpu.sync_copy(rows, acc.at[dst_t.at[j]], add=True)
            return 0
        lax.fori_loop(0, K, chunk, 0)

        plsc.subcore_barrier()
        pltpu.sync_copy(acc.at[pl.ds(s * ZPT, ZPT)],
                        out_hbm.at[c, pl.ds(s * ZPT, ZPT)])

    return hist, agg


def _dinv_body(degp_ref, out_ref):
    d = degp_ref[0:1, :] + degp_ref[1:2, :]
    out_ref[...] = lax.rsqrt(jnp.maximum(d, 1.0))


_dinv_tc = pl.pallas_call(
    _dinv_body,
    out_shape=jax.ShapeDtypeStruct((1, NPAD), jnp.float32),
)

_RB = 400
_G = N // _RB


def _mm1_body(x_ref, w_ref, dinv_ref, out_ref):
    out_ref[...] = dinv_ref[...] * jnp.dot(
        x_ref[...], w_ref[...], preferred_element_type=jnp.float32)


_mm1 = pl.pallas_call(
    _mm1_body,
    grid=(_G,),
    in_specs=[
        pl.BlockSpec((_RB, D), lambda i: (i, 0)),
        pl.BlockSpec((D, D), lambda i: (0, 0)),
        pl.BlockSpec((_RB, 1), lambda i: (i, 0)),
    ],
    out_specs=pl.BlockSpec((_RB, D), lambda i: (i, 0)),
    out_shape=jax.ShapeDtypeStruct((N, D), jnp.float32),
)


def _mid_body(aggp_ref, dinv_ref, b1_ref, w2_ref, out_ref):
    t = (aggp_ref[0] + aggp_ref[1]) * dinv_ref[...] + b1_ref[...]
    h = jnp.maximum(t, 0.0)
    out_ref[...] = dinv_ref[...] * jnp.dot(
        h, w2_ref[...], preferred_element_type=jnp.float32)


_mid = pl.pallas_call(
    _mid_body,
    grid=(_G,),
    in_specs=[
        pl.BlockSpec((NC, _RB, D), lambda i: (0, i, 0)),
        pl.BlockSpec((_RB, 1), lambda i: (i, 0)),
        pl.BlockSpec((1, D), lambda i: (0, 0)),
        pl.BlockSpec((D, D), lambda i: (0, 0)),
    ],
    out_specs=pl.BlockSpec((_RB, D), lambda i: (i, 0)),
    out_shape=jax.ShapeDtypeStruct((N, D), jnp.float32),
)


def _out_body(aggp_ref, dinv_ref, b2_ref, out_ref):
    out_ref[...] = (aggp_ref[0] + aggp_ref[1]) * dinv_ref[...] + b2_ref[...]


_outk = pl.pallas_call(
    _out_body,
    grid=(_G,),
    in_specs=[
        pl.BlockSpec((NC, _RB, D), lambda i: (0, i, 0)),
        pl.BlockSpec((_RB, 1), lambda i: (i, 0)),
        pl.BlockSpec((1, D), lambda i: (0, 0)),
    ],
    out_specs=pl.BlockSpec((_RB, D), lambda i: (i, 0)),
    out_shape=jax.ShapeDtypeStruct((N, D), jnp.float32),
)


def kernel(x, edge_index, W1, b1, W2, b2):
    hist, agg = _sc_kernels()
    src = edge_index[0].astype(jnp.int32)
    dst = edge_index[1].astype(jnp.int32)
    pad = E_PAD - E
    src_r = jnp.concatenate(
        [src, jnp.zeros((pad,), jnp.int32)]).reshape(NW, K, C)
    dst_r = jnp.concatenate(
        [dst, jnp.full((pad,), N, jnp.int32)]).reshape(NW, K, C)

    degp = hist(dst_r)
    dinv = _dinv_tc(degp).reshape(NPAD)[:N].reshape(N, 1)

    p1 = _mm1(x, W1, dinv)
    aggp1 = agg(p1, src_r, dst_r)
    p2 = _mid(aggp1, dinv, b1.reshape(1, D), W2)
    aggp2 = agg(p2, src_r, dst_r)
    return _outk(aggp2, dinv, b2.reshape(1, D))
